# baseline (speedup 1.0000x reference)
import dataclasses
import functools

import jax
import jax.numpy as jnp
from jax import lax
from jax.experimental import pallas as pl
from jax.experimental.pallas import tpu as pltpu
from jax.experimental.pallas import tpu_sc as plsc

N = 10000
E = 320000
D = 128
G = 64

NC = 2
NS = 16
NW = NC * NS
CH = 128
TOTC = E // CH
RC0 = 80
RC1 = 80
RC1_LAST = TOTC - (NS - 1) * (RC0 + RC1) - RC0
ROWS_PER_TILE = 632
NPAD = NS * ROWS_PER_TILE

_mesh = plsc.VectorSubcoreMesh(core_axis_name="c", subcore_axis_name="s")

HN = 10240
HS = HN // NS
HROWS = HS // 128

_GDN = lax.GatherDimensionNumbers(offset_dims=(), collapsed_slice_dims=(0,),
                                  start_index_map=(0,))


def _take16(arr, idx):
    return lax.gather(arr, idx[:, None], _GDN, (1,),
                      mode=lax.GatherScatterMode.PROMISE_IN_BOUNDS)


def _emit_hist(dst_hbm, di, hist, sem, base, n):
    pltpu.async_copy(dst_hbm.at[pl.ds(base, n)], di.at[pl.ds(0, n)],
                     sem).wait()
    iota = lax.iota(jnp.int32, 16)
    idxm1 = jnp.maximum(iota - 1, 0)
    ones = jnp.ones((16,), jnp.float32)

    @pl.loop(0, n)
    def _(j):
        @pl.loop(0, CH // 16)
        def _(c):
            v = di[j, pl.ds(16 * c, 16)]
            ks, _ = plsc.sort_key_val(v, v)
            prev = _take16(ks, idxm1)
            newk = (iota == 0) | (ks != prev)

            def cond(m):
                return jnp.sum(m.astype(jnp.int32), axis=0) > 0

            def body(m):
                mprev = _take16(m.astype(jnp.int32), idxm1)
                f = m & (newk | (mprev == 0))
                plsc.addupdate_scatter(hist, [ks], ones, mask=f)
                return m & (~f)

            lax.while_loop(cond, body, jnp.full((16,), True))


def _sc_degree_body(dst_hbm, out_hbm, hist, di, sh, tmp, res, sem):
    cid = lax.axis_index("c")
    sid = lax.axis_index("s")

    @pl.loop(0, HN // 16)
    def _(i):
        hist[pl.ds(16 * i, 16)] = jnp.zeros((16,), jnp.float32)

    @pl.when(cid == 0)
    def _():
        _emit_hist(dst_hbm, di, hist, sem, sid * (RC0 + RC1), RC0)

    @pl.when((cid == 1) & (sid < NS - 1))
    def _():
        _emit_hist(dst_hbm, di, hist, sem, sid * (RC0 + RC1) + RC0, RC1)

    @pl.when((cid == 1) & (sid == NS - 1))
    def _():
        _emit_hist(dst_hbm, di, hist, sem,
                   (NS - 1) * (RC0 + RC1) + RC0, RC1_LAST)

    pltpu.sync_copy(hist, sh.at[sid])
    plsc.subcore_barrier()

    @pl.loop(0, HROWS * 8)
    def _(i):
        res[i // 8, pl.ds(16 * lax.rem(i, 8), 16)] = \
            jnp.zeros((16,), jnp.float32)

    for k in range(NS):
        pltpu.sync_copy(sh.at[k, pl.ds(sid * HS, HS)], tmp)

        @pl.loop(0, HS // 16)
        def _(i):
            q = 16 * i
            r = q // 128
            cpos = lax.rem(q, 128)
            res[r, pl.ds(cpos, 16)] = res[r, pl.ds(cpos, 16)] \
                + tmp[pl.ds(q, 16)]

    pltpu.sync_copy(res, out_hbm.at[cid, sid])


_cp = pltpu.CompilerParams()
if "needs_layout_passes" in pltpu.CompilerParams.__dataclass_fields__:
    _cp = dataclasses.replace(_cp, needs_layout_passes=False)

_sc_degree = pl.kernel(
    _sc_degree_body,
    out_type=jax.ShapeDtypeStruct((NC, NS, HROWS, 128), jnp.float32),
    compiler_params=_cp,
    mesh=_mesh,
    scratch_types=[
        pltpu.VMEM((HN,), jnp.float32),
        pltpu.VMEM((RC0, CH), jnp.int32),
        pltpu.VMEM_SHARED((NS, HN), jnp.float32),
        pltpu.VMEM((HS,), jnp.float32),
        pltpu.VMEM((HROWS, 128), jnp.float32),
        pltpu.SemaphoreType.DMA,
    ],
)


BLK = 16


def _blocks(rc):
    out = []
    left = rc
    while left > 0:
        out.append(min(BLK, left))
        left -= min(BLK, left)
    return out


def _emit_scatter_pipeline(g_hbm, src_hbm, dst_hbm, acc,
                           sis, dis, buf0, buf1, sem0, sem1, semi,
                           base_chunk, rc):
    sizes = _blocks(rc)
    bases = [base_chunk + BLK * i for i in range(len(sizes))]
    pltpu.async_copy(src_hbm.at[pl.ds(bases[0], sizes[0])],
                     sis[0].at[pl.ds(0, sizes[0])], semi)
    pltpu.async_copy(dst_hbm.at[pl.ds(bases[0], sizes[0])],
                     dis[0].at[pl.ds(0, sizes[0])], semi)
    for b, sz in enumerate(sizes):
        si = sis[b % 2]
        di = dis[b % 2]
        rb = bases[b]
        pltpu.make_async_copy(src_hbm.at[pl.ds(rb, sz)],
                              si.at[pl.ds(0, sz)], semi).wait()
        pltpu.make_async_copy(dst_hbm.at[pl.ds(rb, sz)],
                              di.at[pl.ds(0, sz)], semi).wait()
        if b + 1 < len(sizes):
            nsz = sizes[b + 1]
            pltpu.async_copy(src_hbm.at[pl.ds(rb + BLK, nsz)],
                             sis[(b + 1) % 2].at[pl.ds(0, nsz)], semi)
            pltpu.async_copy(dst_hbm.at[pl.ds(rb + BLK, nsz)],
                             dis[(b + 1) % 2].at[pl.ds(0, nsz)], semi)

        pltpu.async_copy(g_hbm.at[si.at[0]], buf0, sem0)

        @pl.loop(0, sz // 2)
        def _(t):
            j = 2 * t
            pltpu.async_copy(g_hbm.at[si.at[j + 1]], buf1, sem1)
            pltpu.make_async_copy(g_hbm.at[si.at[j]], buf0, sem0).wait()
            pltpu.sync_copy(buf0, acc.at[di.at[j]], add=True)

            @pl.when(j + 2 < sz)
            def _():
                pltpu.async_copy(g_hbm.at[si.at[j + 2]], buf0, sem0)

            pltpu.make_async_copy(g_hbm.at[si.at[j + 1]], buf1, sem1).wait()
            pltpu.sync_copy(buf1, acc.at[di.at[j + 1]], add=True)


def _sc_scatter_body(g_hbm, src_hbm, dst_hbm, zero_hbm, out_hbm,
                     acc, si0, si1, di0, di1, buf0, buf1,
                     sem0, sem1, semi):
    cid = lax.axis_index("c")
    sid = lax.axis_index("s")
    r0 = sid * ROWS_PER_TILE
    pltpu.sync_copy(zero_hbm.at[pl.ds(r0, ROWS_PER_TILE)],
                    acc.at[pl.ds(r0, ROWS_PER_TILE)])
    sis = (si0, si1)
    dis = (di0, di1)
    plsc.subcore_barrier()

    @pl.when(cid == 0)
    def _():
        _emit_scatter_pipeline(g_hbm, src_hbm, dst_hbm, acc, sis, dis,
                               buf0, buf1, sem0, sem1, semi,
                               sid * (RC0 + RC1), RC0)

    @pl.when((cid == 1) & (sid < NS - 1))
    def _():
        _emit_scatter_pipeline(g_hbm, src_hbm, dst_hbm, acc, sis, dis,
                               buf0, buf1, sem0, sem1, semi,
                               sid * (RC0 + RC1) + RC0, RC1)

    @pl.when((cid == 1) & (sid == NS - 1))
    def _():
        _emit_scatter_pipeline(g_hbm, src_hbm, dst_hbm, acc, sis, dis,
                               buf0, buf1, sem0, sem1, semi,
                               (NS - 1) * (RC0 + RC1) + RC0, RC1_LAST)

    plsc.subcore_barrier()
    pltpu.sync_copy(acc.at[pl.ds(r0, ROWS_PER_TILE)],
                    out_hbm.at[cid, pl.ds(r0, ROWS_PER_TILE)])


_sc_scatter = pl.kernel(
    _sc_scatter_body,
    out_type=jax.ShapeDtypeStruct((NC, NPAD, D), jnp.float32),
    mesh=_mesh,
    scratch_types=[
        pltpu.VMEM_SHARED((NPAD, D), jnp.float32),
        pltpu.VMEM((BLK, CH), jnp.int32),
        pltpu.VMEM((BLK, CH), jnp.int32),
        pltpu.VMEM((BLK, CH), jnp.int32),
        pltpu.VMEM((BLK, CH), jnp.int32),
        pltpu.VMEM((CH, D), jnp.float32),
        pltpu.VMEM((CH, D), jnp.float32),
        pltpu.SemaphoreType.DMA,
        pltpu.SemaphoreType.DMA,
        pltpu.SemaphoreType.DMA,
    ],
)


def _dot(a, b, dims):
    return lax.dot_general(a, b, dims, precision=lax.Precision.HIGHEST,
                           preferred_element_type=jnp.float32)


def _mm_body(x_ref, w_ref, o_ref):
    o_ref[...] = _dot(x_ref[...], w_ref[...], (((1,), (0,)), ((), ())))


_tc_mm = pl.pallas_call(
    _mm_body,
    out_shape=jax.ShapeDtypeStruct((N, D), jnp.float32),
)


def _scale_body(degp_ref, h_ref, g_ref, dinv_ref):
    deg = degp_ref[0] + degp_ref[1] + 1.0
    dinv_full = lax.rsqrt(deg).reshape(HN, 1)
    dinv = lax.slice(dinv_full, (0, 0), (N, 1))
    dinv_ref[...] = dinv
    g_ref[...] = h_ref[...] * dinv


_tc_scale = pl.pallas_call(
    _scale_body,
    out_shape=(jax.ShapeDtypeStruct((N, D), jnp.float32),
               jax.ShapeDtypeStruct((N, 1), jnp.float32)),
)


def _mid_body(sp_ref, g_ref, dinv_ref, b_ref, w_ref, gout_ref):
    dinv = dinv_ref[...]
    s = (sp_ref[0, pl.ds(0, N), :] + sp_ref[1, pl.ds(0, N), :]
         + g_ref[...]) * dinv + b_ref[...]
    z = jnp.maximum(s, 0.0)
    h2 = _dot(z, w_ref[...], (((1,), (0,)), ((), ())))
    gout_ref[...] = h2 * dinv


_tc_mid = pl.pallas_call(
    _mid_body,
    out_shape=jax.ShapeDtypeStruct((N, D), jnp.float32),
)


def _final_body(sp_ref, g_ref, dinv_ref, b_ref, batch_ref, o_ref):
    y = (sp_ref[0, pl.ds(0, N), :] + sp_ref[1, pl.ds(0, N), :]
         + g_ref[...]) * dinv_ref[...] + b_ref[...]
    seg = batch_ref[...]
    ids = lax.broadcasted_iota(jnp.int32, (G, N), 0)
    pt = (ids == seg).astype(jnp.float32)
    pooled = _dot(pt, y, (((1,), (0,)), ((), ())))
    cnt = _dot(pt, jnp.ones((N, 1), jnp.float32), (((1,), (0,)), ((), ())))
    mean = pooled / jnp.maximum(cnt, 1.0)
    m = jnp.max(mean, axis=1, keepdims=True)
    ex = jnp.exp(mean - m)
    lse = jnp.log(jnp.sum(ex, axis=1, keepdims=True))
    o_ref[...] = mean - m - lse


_tc_final = pl.pallas_call(
    _final_body,
    out_shape=jax.ShapeDtypeStruct((G, D), jnp.float32),
)


@jax.jit
def _run(x, edge_index, batch, W1, b1, W2, b2):
    dstp = edge_index[1].reshape(TOTC, CH)
    srcp = lax.optimization_barrier(edge_index[0]).reshape(TOTC, CH)
    zeros128 = jnp.zeros((NPAD, D), jnp.float32)

    degp = _sc_degree(dstp).reshape(NC, HN)
    h1 = _tc_mm(x, W1)
    g1, dinv = _tc_scale(degp, h1)
    s1 = _sc_scatter(g1, srcp, dstp, zeros128)
    g2 = _tc_mid(s1, g1, dinv, b1.reshape(1, D), W2)
    s2 = _sc_scatter(g2, srcp, dstp, zeros128)
    return _tc_final(s2, g2, dinv, b2.reshape(1, D), batch.reshape(1, N))


def kernel(x, edge_index, batch, W1, b1, W2, b2):
    return _run(x, edge_index, batch, W1, b1, W2, b2)


# speedup vs baseline: 1.0309x; 1.0309x over previous
import dataclasses
import functools

import jax
import jax.numpy as jnp
from jax import lax
from jax.experimental import pallas as pl
from jax.experimental.pallas import tpu as pltpu
from jax.experimental.pallas import tpu_sc as plsc

N = 10000
E = 320000
D = 128
G = 64

NC = 2
NS = 16
NW = NC * NS
CH = 128
TOTC = E // CH
RC0 = 80
RC1 = 80
RC1_LAST = TOTC - (NS - 1) * (RC0 + RC1) - RC0
ROWS_PER_TILE = 632
NPAD = NS * ROWS_PER_TILE

_mesh = plsc.VectorSubcoreMesh(core_axis_name="c", subcore_axis_name="s")

HN = 10240
HS = HN // NS
HROWS = HS // 128

_GDN = lax.GatherDimensionNumbers(offset_dims=(), collapsed_slice_dims=(0,),
                                  start_index_map=(0,))


def _take16(arr, idx):
    return lax.gather(arr, idx[:, None], _GDN, (1,),
                      mode=lax.GatherScatterMode.PROMISE_IN_BOUNDS)


def _emit_hist(ei_hbm, di, hist, sem, base, n):
    pltpu.async_copy(ei_hbm.at[1, pl.ds(base * CH, n * CH)],
                     di.at[pl.ds(0, n * CH)], sem).wait()
    iota = lax.iota(jnp.int32, 16)
    idxm1 = jnp.maximum(iota - 1, 0)
    ones = jnp.ones((16,), jnp.float32)

    @pl.loop(0, n)
    def _(j):
        @pl.loop(0, CH // 16)
        def _(c):
            v = di[pl.ds(j * CH + 16 * c, 16)]
            ks, _ = plsc.sort_key_val(v, v)
            prev = _take16(ks, idxm1)
            newk = (iota == 0) | (ks != prev)

            def cond(m):
                return jnp.sum(m.astype(jnp.int32), axis=0) > 0

            def body(m):
                mprev = _take16(m.astype(jnp.int32), idxm1)
                f = m & (newk | (mprev == 0))
                plsc.addupdate_scatter(hist, [ks], ones, mask=f)
                return m & (~f)

            lax.while_loop(cond, body, jnp.full((16,), True))


def _sc_degree_body(ei_hbm, out_hbm, hist, di, sh, tmp, res, sem):
    cid = lax.axis_index("c")
    sid = lax.axis_index("s")

    @pl.loop(0, HN // 16)
    def _(i):
        hist[pl.ds(16 * i, 16)] = jnp.zeros((16,), jnp.float32)

    @pl.when(cid == 0)
    def _():
        _emit_hist(ei_hbm, di, hist, sem, sid * (RC0 + RC1), RC0)

    @pl.when((cid == 1) & (sid < NS - 1))
    def _():
        _emit_hist(ei_hbm, di, hist, sem, sid * (RC0 + RC1) + RC0, RC1)

    @pl.when((cid == 1) & (sid == NS - 1))
    def _():
        _emit_hist(ei_hbm, di, hist, sem,
                   (NS - 1) * (RC0 + RC1) + RC0, RC1_LAST)

    pltpu.sync_copy(hist, sh.at[sid])
    plsc.subcore_barrier()

    @pl.loop(0, HROWS * 8)
    def _(i):
        res[i // 8, pl.ds(16 * lax.rem(i, 8), 16)] = \
            jnp.zeros((16,), jnp.float32)

    for k in range(NS):
        pltpu.sync_copy(sh.at[k, pl.ds(sid * HS, HS)], tmp)

        @pl.loop(0, HS // 16)
        def _(i):
            q = 16 * i
            r = q // 128
            cpos = lax.rem(q, 128)
            res[r, pl.ds(cpos, 16)] = res[r, pl.ds(cpos, 16)] \
                + tmp[pl.ds(q, 16)]

    pltpu.sync_copy(res, out_hbm.at[cid, sid])


_cp = pltpu.CompilerParams()
if "needs_layout_passes" in pltpu.CompilerParams.__dataclass_fields__:
    _cp = dataclasses.replace(_cp, needs_layout_passes=False)

_sc_degree = pl.kernel(
    _sc_degree_body,
    out_type=jax.ShapeDtypeStruct((NC, NS, HROWS, 128), jnp.float32),
    compiler_params=_cp,
    mesh=_mesh,
    scratch_types=[
        pltpu.VMEM((HN,), jnp.float32),
        pltpu.VMEM((RC0 * CH,), jnp.int32),
        pltpu.VMEM_SHARED((NS, HN), jnp.float32),
        pltpu.VMEM((HS,), jnp.float32),
        pltpu.VMEM((HROWS, 128), jnp.float32),
        pltpu.SemaphoreType.DMA,
    ],
)


BLK = 16


def _blocks(rc):
    out = []
    left = rc
    while left > 0:
        out.append(min(BLK, left))
        left -= min(BLK, left)
    return out


def _emit_scatter_pipeline(g_hbm, ei_hbm, acc,
                           sis, dis, di2d, buf0, buf1, sem0, sem1, semi,
                           base_chunk, rc):
    sizes = _blocks(rc)
    bases = [base_chunk + BLK * i for i in range(len(sizes))]
    pltpu.async_copy(ei_hbm.at[0, pl.ds(bases[0] * CH, sizes[0] * CH)],
                     sis[0].at[pl.ds(0, sizes[0] * CH)], semi)
    pltpu.async_copy(ei_hbm.at[1, pl.ds(bases[0] * CH, sizes[0] * CH)],
                     dis[0].at[pl.ds(0, sizes[0] * CH)], semi)
    for b, sz in enumerate(sizes):
        si = sis[b % 2]
        di = dis[b % 2]
        rb = bases[b]
        pltpu.make_async_copy(ei_hbm.at[0, pl.ds(rb * CH, sz * CH)],
                              si.at[pl.ds(0, sz * CH)], semi).wait()
        pltpu.make_async_copy(ei_hbm.at[1, pl.ds(rb * CH, sz * CH)],
                              di.at[pl.ds(0, sz * CH)], semi).wait()
        if b + 1 < len(sizes):
            nsz = sizes[b + 1]
            pltpu.async_copy(
                ei_hbm.at[0, pl.ds((rb + BLK) * CH, nsz * CH)],
                sis[(b + 1) % 2].at[pl.ds(0, nsz * CH)], semi)
            pltpu.async_copy(
                ei_hbm.at[1, pl.ds((rb + BLK) * CH, nsz * CH)],
                dis[(b + 1) % 2].at[pl.ds(0, nsz * CH)], semi)

        @pl.loop(0, sz * 8)
        def _(i):
            di2d[i // 8, pl.ds(16 * lax.rem(i, 8), 16)] = \
                di[pl.ds(16 * i, 16)]

        pltpu.async_copy(g_hbm.at[si.at[pl.ds(0, CH)]], buf0, sem0)

        @pl.loop(0, sz // 2)
        def _(t):
            j = 2 * t
            pltpu.async_copy(g_hbm.at[si.at[pl.ds((j + 1) * CH, CH)]],
                             buf1, sem1)
            pltpu.make_async_copy(g_hbm.at[si.at[pl.ds(j * CH, CH)]],
                                  buf0, sem0).wait()
            pltpu.sync_copy(buf0, acc.at[di2d.at[j]], add=True)

            @pl.when(j + 2 < sz)
            def _():
                pltpu.async_copy(g_hbm.at[si.at[pl.ds((j + 2) * CH, CH)]],
                                 buf0, sem0)

            pltpu.make_async_copy(g_hbm.at[si.at[pl.ds((j + 1) * CH, CH)]],
                                  buf1, sem1).wait()
            pltpu.sync_copy(buf1, acc.at[di2d.at[j + 1]], add=True)


def _sc_scatter_body(g_hbm, ei_hbm, zero_hbm, out_hbm,
                     acc, si0, si1, di0, di1, di2d, buf0, buf1,
                     sem0, sem1, semi):
    cid = lax.axis_index("c")
    sid = lax.axis_index("s")
    r0 = sid * ROWS_PER_TILE
    pltpu.sync_copy(zero_hbm.at[pl.ds(r0, ROWS_PER_TILE)],
                    acc.at[pl.ds(r0, ROWS_PER_TILE)])
    sis = (si0, si1)
    dis = (di0, di1)
    plsc.subcore_barrier()

    @pl.when(cid == 0)
    def _():
        _emit_scatter_pipeline(g_hbm, ei_hbm, acc, sis, dis, di2d,
                               buf0, buf1, sem0, sem1, semi,
                               sid * (RC0 + RC1), RC0)

    @pl.when((cid == 1) & (sid < NS - 1))
    def _():
        _emit_scatter_pipeline(g_hbm, ei_hbm, acc, sis, dis, di2d,
                               buf0, buf1, sem0, sem1, semi,
                               sid * (RC0 + RC1) + RC0, RC1)

    @pl.when((cid == 1) & (sid == NS - 1))
    def _():
        _emit_scatter_pipeline(g_hbm, ei_hbm, acc, sis, dis, di2d,
                               buf0, buf1, sem0, sem1, semi,
                               (NS - 1) * (RC0 + RC1) + RC0, RC1_LAST)

    plsc.subcore_barrier()
    pltpu.sync_copy(acc.at[pl.ds(r0, ROWS_PER_TILE)],
                    out_hbm.at[cid, pl.ds(r0, ROWS_PER_TILE)])


_sc_scatter = pl.kernel(
    _sc_scatter_body,
    out_type=jax.ShapeDtypeStruct((NC, NPAD, D), jnp.float32),
    mesh=_mesh,
    scratch_types=[
        pltpu.VMEM_SHARED((NPAD, D), jnp.float32),
        pltpu.VMEM((BLK * CH,), jnp.int32),
        pltpu.VMEM((BLK * CH,), jnp.int32),
        pltpu.VMEM((BLK * CH,), jnp.int32),
        pltpu.VMEM((BLK * CH,), jnp.int32),
        pltpu.VMEM((BLK, CH), jnp.int32),
        pltpu.VMEM((CH, D), jnp.float32),
        pltpu.VMEM((CH, D), jnp.float32),
        pltpu.SemaphoreType.DMA,
        pltpu.SemaphoreType.DMA,
        pltpu.SemaphoreType.DMA,
    ],
)


def _dot(a, b, dims):
    return lax.dot_general(a, b, dims, precision=lax.Precision.HIGHEST,
                           preferred_element_type=jnp.float32)


def _mm_body(x_ref, w_ref, o_ref):
    o_ref[...] = _dot(x_ref[...], w_ref[...], (((1,), (0,)), ((), ())))


_tc_mm = pl.pallas_call(
    _mm_body,
    out_shape=jax.ShapeDtypeStruct((N, D), jnp.float32),
)


def _scale_body(degp_ref, h_ref, g_ref, dinv_ref):
    deg = degp_ref[0] + degp_ref[1] + 1.0
    dinv_full = lax.rsqrt(deg).reshape(HN, 1)
    dinv = lax.slice(dinv_full, (0, 0), (N, 1))
    dinv_ref[...] = dinv
    g_ref[...] = h_ref[...] * dinv


_tc_scale = pl.pallas_call(
    _scale_body,
    out_shape=(jax.ShapeDtypeStruct((N, D), jnp.float32),
               jax.ShapeDtypeStruct((N, 1), jnp.float32)),
)


def _mid_body(sp_ref, g_ref, dinv_ref, b_ref, w_ref, gout_ref):
    dinv = dinv_ref[...]
    s = (sp_ref[0, pl.ds(0, N), :] + sp_ref[1, pl.ds(0, N), :]
         + g_ref[...]) * dinv + b_ref[...]
    z = jnp.maximum(s, 0.0)
    h2 = _dot(z, w_ref[...], (((1,), (0,)), ((), ())))
    gout_ref[...] = h2 * dinv


_tc_mid = pl.pallas_call(
    _mid_body,
    out_shape=jax.ShapeDtypeStruct((N, D), jnp.float32),
)


def _final_body(sp_ref, g_ref, dinv_ref, b_ref, batch_ref, o_ref):
    y = (sp_ref[0, pl.ds(0, N), :] + sp_ref[1, pl.ds(0, N), :]
         + g_ref[...]) * dinv_ref[...] + b_ref[...]
    seg = batch_ref[...]
    ids = lax.broadcasted_iota(jnp.int32, (G, N), 0)
    pt = (ids == seg).astype(jnp.float32)
    pooled = _dot(pt, y, (((1,), (0,)), ((), ())))
    cnt = _dot(pt, jnp.ones((N, 1), jnp.float32), (((1,), (0,)), ((), ())))
    mean = pooled / jnp.maximum(cnt, 1.0)
    m = jnp.max(mean, axis=1, keepdims=True)
    ex = jnp.exp(mean - m)
    lse = jnp.log(jnp.sum(ex, axis=1, keepdims=True))
    o_ref[...] = mean - m - lse


_tc_final = pl.pallas_call(
    _final_body,
    out_shape=jax.ShapeDtypeStruct((G, D), jnp.float32),
)


@jax.jit
def _run(x, edge_index, batch, W1, b1, W2, b2):
    zeros128 = jnp.zeros((NPAD, D), jnp.float32)

    degp = _sc_degree(edge_index).reshape(NC, HN)
    h1 = _tc_mm(x, W1)
    g1, dinv = _tc_scale(degp, h1)
    s1 = _sc_scatter(g1, edge_index, zeros128)
    g2 = _tc_mid(s1, g1, dinv, b1.reshape(1, D), W2)
    s2 = _sc_scatter(g2, edge_index, zeros128)
    return _tc_final(s2, g2, dinv, b2.reshape(1, D), batch.reshape(1, N))


def kernel(x, edge_index, batch, W1, b1, W2, b2):
    return _run(x, edge_index, batch, W1, b1, W2, b2)


# speedup vs baseline: 1.0438x; 1.0125x over previous
import dataclasses
import functools

import jax
import jax.numpy as jnp
from jax import lax
from jax.experimental import pallas as pl
from jax.experimental.pallas import tpu as pltpu
from jax.experimental.pallas import tpu_sc as plsc

N = 10000
E = 320000
D = 128
G = 64

NC = 2
NS = 16
NW = NC * NS
CH = 128
TOTC = E // CH
RC0 = 80
RC1 = 80
RC1_LAST = TOTC - (NS - 1) * (RC0 + RC1) - RC0
ROWS_PER_TILE = 632
NPAD = NS * ROWS_PER_TILE

_mesh = plsc.VectorSubcoreMesh(core_axis_name="c", subcore_axis_name="s")

HN = 10240
HS = HN // NS
HROWS = HS // 128

_GDN = lax.GatherDimensionNumbers(offset_dims=(), collapsed_slice_dims=(0,),
                                  start_index_map=(0,))


def _take16(arr, idx):
    return lax.gather(arr, idx[:, None], _GDN, (1,),
                      mode=lax.GatherScatterMode.PROMISE_IN_BOUNDS)


def _emit_hist(ei_hbm, di, hist, sem, base, n):
    pltpu.async_copy(ei_hbm.at[1, pl.ds(base * CH, n * CH)],
                     di.at[pl.ds(0, n * CH)], sem).wait()
    iota = lax.iota(jnp.int32, 16)
    idxm1 = jnp.maximum(iota - 1, 0)
    ones = jnp.ones((16,), jnp.float32)

    @pl.loop(0, n)
    def _(j):
        @pl.loop(0, CH // 16)
        def _(c):
            v = di[pl.ds(j * CH + 16 * c, 16)]
            ks, _ = plsc.sort_key_val(v, v)
            prev = _take16(ks, idxm1)
            newk = (iota == 0) | (ks != prev)
            plsc.addupdate_scatter(hist, [ks], ones, mask=newk)
            dups = ~newk

            @pl.when(jnp.sum(dups.astype(jnp.int32), axis=0) > 0)
            def _():
                def cond(m):
                    return jnp.sum(m.astype(jnp.int32), axis=0) > 0

                def body(m):
                    mprev = _take16(m.astype(jnp.int32), idxm1)
                    f = m & (newk | (mprev == 0))
                    plsc.addupdate_scatter(hist, [ks], ones, mask=f)
                    return m & (~f)

                lax.while_loop(cond, body, dups)


def _sc_degree_body(ei_hbm, out_hbm, hist, di, sh, tmp, res, sem):
    cid = lax.axis_index("c")
    sid = lax.axis_index("s")

    @pl.loop(0, HN // 16)
    def _(i):
        hist[pl.ds(16 * i, 16)] = jnp.zeros((16,), jnp.float32)

    @pl.when(cid == 0)
    def _():
        _emit_hist(ei_hbm, di, hist, sem, sid * (RC0 + RC1), RC0)

    @pl.when((cid == 1) & (sid < NS - 1))
    def _():
        _emit_hist(ei_hbm, di, hist, sem, sid * (RC0 + RC1) + RC0, RC1)

    @pl.when((cid == 1) & (sid == NS - 1))
    def _():
        _emit_hist(ei_hbm, di, hist, sem,
                   (NS - 1) * (RC0 + RC1) + RC0, RC1_LAST)

    pltpu.sync_copy(hist, sh.at[sid])
    plsc.subcore_barrier()

    @pl.loop(0, HROWS * 8)
    def _(i):
        res[i // 8, pl.ds(16 * lax.rem(i, 8), 16)] = \
            jnp.zeros((16,), jnp.float32)

    pltpu.sync_copy(sh.at[:, pl.ds(sid * HS, HS)], tmp)
    for k in range(NS):
        @pl.loop(0, HS // 16)
        def _(i):
            q = 16 * i
            r = q // 128
            cpos = lax.rem(q, 128)
            res[r, pl.ds(cpos, 16)] = res[r, pl.ds(cpos, 16)] \
                + tmp[k, pl.ds(q, 16)]

    pltpu.sync_copy(res, out_hbm.at[cid, sid])


_cp = pltpu.CompilerParams()
if "needs_layout_passes" in pltpu.CompilerParams.__dataclass_fields__:
    _cp = dataclasses.replace(_cp, needs_layout_passes=False)

_sc_degree = pl.kernel(
    _sc_degree_body,
    out_type=jax.ShapeDtypeStruct((NC, NS, HROWS, 128), jnp.float32),
    compiler_params=_cp,
    mesh=_mesh,
    scratch_types=[
        pltpu.VMEM((HN,), jnp.float32),
        pltpu.VMEM((RC0 * CH,), jnp.int32),
        pltpu.VMEM_SHARED((NS, HN), jnp.float32),
        pltpu.VMEM((NS, HS), jnp.float32),
        pltpu.VMEM((HROWS, 128), jnp.float32),
        pltpu.SemaphoreType.DMA,
    ],
)


BLK = 16


def _blocks(rc):
    out = []
    left = rc
    while left > 0:
        out.append(min(BLK, left))
        left -= min(BLK, left)
    return out


def _emit_scatter_pipeline(g_hbm, ei_hbm, acc,
                           sis, dis, di2d, buf0, buf1, sem0, sem1, semi,
                           base_chunk, rc):
    sizes = _blocks(rc)
    bases = [base_chunk + BLK * i for i in range(len(sizes))]
    pltpu.async_copy(ei_hbm.at[0, pl.ds(bases[0] * CH, sizes[0] * CH)],
                     sis[0].at[pl.ds(0, sizes[0] * CH)], semi)
    pltpu.async_copy(ei_hbm.at[1, pl.ds(bases[0] * CH, sizes[0] * CH)],
                     dis[0].at[pl.ds(0, sizes[0] * CH)], semi)
    for b, sz in enumerate(sizes):
        si = sis[b % 2]
        di = dis[b % 2]
        rb = bases[b]
        pltpu.make_async_copy(ei_hbm.at[0, pl.ds(rb * CH, sz * CH)],
                              si.at[pl.ds(0, sz * CH)], semi).wait()
        pltpu.make_async_copy(ei_hbm.at[1, pl.ds(rb * CH, sz * CH)],
                              di.at[pl.ds(0, sz * CH)], semi).wait()
        if b + 1 < len(sizes):
            nsz = sizes[b + 1]
            pltpu.async_copy(
                ei_hbm.at[0, pl.ds((rb + BLK) * CH, nsz * CH)],
                sis[(b + 1) % 2].at[pl.ds(0, nsz * CH)], semi)
            pltpu.async_copy(
                ei_hbm.at[1, pl.ds((rb + BLK) * CH, nsz * CH)],
                dis[(b + 1) % 2].at[pl.ds(0, nsz * CH)], semi)

        @pl.loop(0, sz * 8)
        def _(i):
            di2d[i // 8, pl.ds(16 * lax.rem(i, 8), 16)] = \
                di[pl.ds(16 * i, 16)]

        pltpu.async_copy(g_hbm.at[si.at[pl.ds(0, CH)]], buf0, sem0)

        @pl.loop(0, sz // 2)
        def _(t):
            j = 2 * t
            pltpu.async_copy(g_hbm.at[si.at[pl.ds((j + 1) * CH, CH)]],
                             buf1, sem1)
            pltpu.make_async_copy(g_hbm.at[si.at[pl.ds(j * CH, CH)]],
                                  buf0, sem0).wait()
            pltpu.sync_copy(buf0, acc.at[di2d.at[j]], add=True)

            @pl.when(j + 2 < sz)
            def _():
                pltpu.async_copy(g_hbm.at[si.at[pl.ds((j + 2) * CH, CH)]],
                                 buf0, sem0)

            pltpu.make_async_copy(g_hbm.at[si.at[pl.ds((j + 1) * CH, CH)]],
                                  buf1, sem1).wait()
            pltpu.sync_copy(buf1, acc.at[di2d.at[j + 1]], add=True)


def _sc_scatter_body(g_hbm, ei_hbm, zero_hbm, out_hbm,
                     acc, si0, si1, di0, di1, di2d, buf0, buf1,
                     sem0, sem1, semi):
    cid = lax.axis_index("c")
    sid = lax.axis_index("s")
    r0 = sid * ROWS_PER_TILE
    pltpu.sync_copy(zero_hbm.at[pl.ds(r0, ROWS_PER_TILE)],
                    acc.at[pl.ds(r0, ROWS_PER_TILE)])
    sis = (si0, si1)
    dis = (di0, di1)
    plsc.subcore_barrier()

    @pl.when(cid == 0)
    def _():
        _emit_scatter_pipeline(g_hbm, ei_hbm, acc, sis, dis, di2d,
                               buf0, buf1, sem0, sem1, semi,
                               sid * (RC0 + RC1), RC0)

    @pl.when((cid == 1) & (sid < NS - 1))
    def _():
        _emit_scatter_pipeline(g_hbm, ei_hbm, acc, sis, dis, di2d,
                               buf0, buf1, sem0, sem1, semi,
                               sid * (RC0 + RC1) + RC0, RC1)

    @pl.when((cid == 1) & (sid == NS - 1))
    def _():
        _emit_scatter_pipeline(g_hbm, ei_hbm, acc, sis, dis, di2d,
                               buf0, buf1, sem0, sem1, semi,
                               (NS - 1) * (RC0 + RC1) + RC0, RC1_LAST)

    plsc.subcore_barrier()
    pltpu.sync_copy(acc.at[pl.ds(r0, ROWS_PER_TILE)],
                    out_hbm.at[cid, pl.ds(r0, ROWS_PER_TILE)])


_sc_scatter = pl.kernel(
    _sc_scatter_body,
    out_type=jax.ShapeDtypeStruct((NC, NPAD, D), jnp.float32),
    mesh=_mesh,
    scratch_types=[
        pltpu.VMEM_SHARED((NPAD, D), jnp.float32),
        pltpu.VMEM((BLK * CH,), jnp.int32),
        pltpu.VMEM((BLK * CH,), jnp.int32),
        pltpu.VMEM((BLK * CH,), jnp.int32),
        pltpu.VMEM((BLK * CH,), jnp.int32),
        pltpu.VMEM((BLK, CH), jnp.int32),
        pltpu.VMEM((CH, D), jnp.float32),
        pltpu.VMEM((CH, D), jnp.float32),
        pltpu.SemaphoreType.DMA,
        pltpu.SemaphoreType.DMA,
        pltpu.SemaphoreType.DMA,
    ],
)


def _dot(a, b, dims):
    return lax.dot_general(a, b, dims, precision=lax.Precision.HIGHEST,
                           preferred_element_type=jnp.float32)


def _mm_body(x_ref, w_ref, o_ref):
    o_ref[...] = _dot(x_ref[...], w_ref[...], (((1,), (0,)), ((), ())))


_tc_mm = pl.pallas_call(
    _mm_body,
    out_shape=jax.ShapeDtypeStruct((N, D), jnp.float32),
)


def _scale_body(degp_ref, h_ref, g_ref, dinv_ref):
    deg = degp_ref[0] + degp_ref[1] + 1.0
    dinv_full = lax.rsqrt(deg).reshape(HN, 1)
    dinv = lax.slice(dinv_full, (0, 0), (N, 1))
    dinv_ref[...] = dinv
    g_ref[...] = h_ref[...] * dinv


_tc_scale = pl.pallas_call(
    _scale_body,
    out_shape=(jax.ShapeDtypeStruct((N, D), jnp.float32),
               jax.ShapeDtypeStruct((N, 1), jnp.float32)),
)


def _mid_body(sp_ref, g_ref, dinv_ref, b_ref, w_ref, gout_ref):
    dinv = dinv_ref[...]
    s = (sp_ref[0, pl.ds(0, N), :] + sp_ref[1, pl.ds(0, N), :]
         + g_ref[...]) * dinv + b_ref[...]
    z = jnp.maximum(s, 0.0)
    h2 = _dot(z, w_ref[...], (((1,), (0,)), ((), ())))
    gout_ref[...] = h2 * dinv


_tc_mid = pl.pallas_call(
    _mid_body,
    out_shape=jax.ShapeDtypeStruct((N, D), jnp.float32),
)


def _final_body(sp_ref, g_ref, dinv_ref, b_ref, batch_ref, o_ref):
    y = (sp_ref[0, pl.ds(0, N), :] + sp_ref[1, pl.ds(0, N), :]
         + g_ref[...]) * dinv_ref[...] + b_ref[...]
    seg = batch_ref[...]
    ids = lax.broadcasted_iota(jnp.int32, (G, N), 0)
    pt = (ids == seg).astype(jnp.float32)
    pooled = _dot(pt, y, (((1,), (0,)), ((), ())))
    cnt = _dot(pt, jnp.ones((N, 1), jnp.float32), (((1,), (0,)), ((), ())))
    mean = pooled / jnp.maximum(cnt, 1.0)
    m = jnp.max(mean, axis=1, keepdims=True)
    ex = jnp.exp(mean - m)
    lse = jnp.log(jnp.sum(ex, axis=1, keepdims=True))
    o_ref[...] = mean - m - lse


_tc_final = pl.pallas_call(
    _final_body,
    out_shape=jax.ShapeDtypeStruct((G, D), jnp.float32),
)


@jax.jit
def _run(x, edge_index, batch, W1, b1, W2, b2):
    zeros128 = jnp.zeros((NPAD, D), jnp.float32)

    degp = _sc_degree(edge_index).reshape(NC, HN)
    h1 = _tc_mm(x, W1)
    g1, dinv = _tc_scale(degp, h1)
    s1 = _sc_scatter(g1, edge_index, zeros128)
    g2 = _tc_mid(s1, g1, dinv, b1.reshape(1, D), W2)
    s2 = _sc_scatter(g2, edge_index, zeros128)
    return _tc_final(s2, g2, dinv, b2.reshape(1, D), batch.reshape(1, N))


def kernel(x, edge_index, batch, W1, b1, W2, b2):
    return _run(x, edge_index, batch, W1, b1, W2, b2)
